# Initial kernel scaffold; baseline (speedup 1.0000x reference)
import jax
import jax.numpy as jnp
from jax.experimental import pallas as pl


def kernel(gate_logits):
    raise NotImplementedError("write your pallas kernel here")


# speedup vs baseline: 1.4289x; 1.4289x over previous
import functools

import numpy as np
import jax
import jax.numpy as jnp
from jax import lax
from jax.experimental import pallas as pl
from jax.experimental.pallas import tpu as pltpu
from jax.experimental.pallas import tpu_sc as plsc

E = 64
L = 16
N_TOK = 131072
NC, NS = 2, 16
W = NC * NS
TW = N_TOK // W
C = 512
CE = C * E
NCHUNK = TW // C
G = C // L


def _bit(e):
    v = 1 << (e % 32)
    return v - (1 << 32) if v >= (1 << 31) else v


_mesh = plsc.VectorSubcoreMesh(
    core_axis_name="c", subcore_axis_name="s", num_cores=NC, num_subcores=NS
)


@functools.partial(
    pl.kernel,
    out_type=(
        jax.ShapeDtypeStruct((W, E, L), jnp.float32),
        jax.ShapeDtypeStruct((W, 2, L), jnp.int32),
    ),
    mesh=_mesh,
    compiler_params=pltpu.CompilerParams(
        needs_layout_passes=False, use_tc_tiling_on_sc=False
    ),
    scratch_types=[
        pltpu.VMEM((CE,), jnp.float32),
        pltpu.VMEM((E, L), jnp.float32),
        pltpu.VMEM((E, L), jnp.float32),
        pltpu.VMEM((2, L), jnp.int32),
    ],
)
def _sc_balance(gate_hbm, w_out, hit_out, buf, evbuf, accw, hit_scr):
    wid = lax.axis_index("s") * NC + lax.axis_index("c")
    base = wid * (TW * E)
    iota = lax.iota(jnp.int32, L)
    zero_v = jnp.zeros((L,), jnp.float32)
    neg_inf = jnp.full((L,), -jnp.inf, dtype=jnp.float32)
    zmask = jnp.zeros((L,), jnp.int32)

    for e in range(E):
        accw[e, :] = zero_v

    def chunk_body(c, hit_carry):
        pltpu.sync_copy(gate_hbm.at[pl.ds(base + c * CE, CE)], buf)

        def group_body(g, hc):
            hlo, hhi = hc
            idx0 = g * (L * E) + iota * E
            s = zero_v
            m1 = neg_inf
            m2 = neg_inf
            for e in range(E):
                v = plsc.load_gather(buf, [idx0 + e])
                ev = jnp.exp(v)
                evbuf[e, :] = ev
                s = s + ev
                m2 = jnp.maximum(m2, jnp.minimum(m1, ev))
                m1 = jnp.maximum(m1, ev)
            r = 1.0 / s
            for e in range(E):
                ev = evbuf[e, :]
                accw[e, :] = accw[e, :] + ev * r
                h = ev >= m2
                if e < 32:
                    hlo = jnp.where(h, hlo | _bit(e), hlo)
                else:
                    hhi = jnp.where(h, hhi | _bit(e), hhi)
            return (hlo, hhi)

        return lax.fori_loop(0, G, group_body, hit_carry)

    hlo, hhi = lax.fori_loop(0, NCHUNK, chunk_body, (zmask, zmask))
    hit_scr[0, :] = hlo
    hit_scr[1, :] = hhi
    pltpu.sync_copy(accw, w_out.at[wid])
    pltpu.sync_copy(hit_scr, hit_out.at[wid])


def kernel(gate_logits):
    w_parts, hit_parts = _sc_balance(gate_logits.reshape(-1))
    total_w = jnp.sum(w_parts)
    u = lax.bitcast_convert_type(hit_parts, jnp.uint32)
    shifts = jnp.arange(32, dtype=jnp.uint32)
    bits = (u[:, :, :, None] >> shifts[None, None, None, :]) & np.uint32(1)
    hit_any = bits.astype(jnp.bool_).any(axis=(0, 2)).reshape(-1)
    tpe = jnp.mean(hit_any.astype(jnp.float32))
    return tpe * (total_w / np.float32(N_TOK)) * np.float32(E)


# speedup vs baseline: 2.4617x; 1.7228x over previous
import functools

import numpy as np
import jax
import jax.numpy as jnp
from jax import lax
from jax.experimental import pallas as pl
from jax.experimental.pallas import tpu as pltpu
from jax.experimental.pallas import tpu_sc as plsc

E = 64
L = 16
N_TOK = 131072
NC, NS = 2, 16
W = NC * NS
TW = N_TOK // W
C = 512
CE = C * E
NCHUNK = TW // C
G = C // L

_mesh = plsc.VectorSubcoreMesh(
    core_axis_name="c", subcore_axis_name="s", num_cores=NC, num_subcores=NS
)


@functools.partial(
    pl.kernel,
    out_type=(
        jax.ShapeDtypeStruct((W, E, L), jnp.float32),
        jax.ShapeDtypeStruct((W, E, L), jnp.float32),
    ),
    mesh=_mesh,
    compiler_params=pltpu.CompilerParams(
        needs_layout_passes=False, use_tc_tiling_on_sc=False
    ),
    scratch_types=[
        pltpu.VMEM((CE,), jnp.float32),
        pltpu.VMEM((CE,), jnp.float32),
        pltpu.VMEM((E, L), jnp.float32),
        pltpu.VMEM((E, L), jnp.float32),
        pltpu.VMEM((E, L), jnp.float32),
        pltpu.SemaphoreType.DMA,
        pltpu.SemaphoreType.DMA,
    ],
)
def _sc_balance(gate_hbm, w_out, hit_out, buf0, buf1, evbuf, accw, acchit,
                sem0, sem1):
    wid = lax.axis_index("s") * NC + lax.axis_index("c")
    base = wid * (TW * E)
    iota = lax.iota(jnp.int32, L)
    zero_v = jnp.zeros((L,), jnp.float32)
    neg_inf = jnp.full((L,), -jnp.inf, dtype=jnp.float32)
    bufs = (buf0, buf1)
    sems = (sem0, sem1)

    def chunk_copy(c_idx, b):
        off = base + jnp.minimum(c_idx, NCHUNK - 1) * CE
        return pltpu.make_async_copy(
            gate_hbm.at[pl.ds(off, CE)], bufs[b], sems[b]
        )

    @plsc.parallel_loop(0, E, 1, unroll=8)
    def _init(e):
        accw[e, :] = zero_v
        acchit[e, :] = zero_v

    chunk_copy(0, 0).start()
    chunk_copy(1, 1).start()

    def process(buf):
        def group_body(g, _):
            idx0 = g * (L * E) + iota * E

            @plsc.parallel_loop(0, E, 1, unroll=8, carry=(zero_v, neg_inf, neg_inf))
            def p1(e, carry):
                s, m1, m2 = carry
                v = plsc.load_gather(buf, [idx0 + e])
                ev = jnp.exp(v)
                evbuf[e, :] = ev
                m2 = jnp.maximum(m2, jnp.minimum(m1, ev))
                m1 = jnp.maximum(m1, ev)
                return s + ev, m1, m2

            s, _, m2 = p1
            r = 1.0 / s

            @plsc.parallel_loop(0, E, 1, unroll=8)
            def _p2(e):
                ev = evbuf[e, :]
                accw[e, :] = accw[e, :] + ev * r
                h = jnp.where(ev >= m2, 1.0, 0.0).astype(jnp.float32)
                acchit[e, :] = jnp.maximum(acchit[e, :], h)

            return 0

        lax.fori_loop(0, G, group_body, 0)

    def chunk_body(cc, _):
        for b in range(2):
            c = cc * 2 + b
            chunk_copy(c, b).wait()
            process(bufs[b])
            chunk_copy(c + 2, b).start()
        return 0

    lax.fori_loop(0, NCHUNK // 2, chunk_body, 0)
    chunk_copy(NCHUNK, 0).wait()
    chunk_copy(NCHUNK + 1, 1).wait()

    pltpu.sync_copy(accw, w_out.at[wid])
    pltpu.sync_copy(acchit, hit_out.at[wid])


def kernel(gate_logits):
    w_parts, hit_parts = _sc_balance(gate_logits.reshape(-1))
    total_w = jnp.sum(w_parts)
    hit_any = jnp.max(hit_parts, axis=(0, 2)) > 0.5
    tpe = jnp.mean(hit_any.astype(jnp.float32))
    return tpe * (total_w / np.float32(N_TOK)) * np.float32(E)


# speedup vs baseline: 3.0403x; 1.2350x over previous
import functools

import numpy as np
import jax
import jax.numpy as jnp
from jax import lax
from jax.experimental import pallas as pl
from jax.experimental.pallas import tpu as pltpu
from jax.experimental.pallas import tpu_sc as plsc

E = 64
L = 16
N_TOK = 131072
NC, NS = 2, 16
W = NC * NS
TW = N_TOK // W
C = 512
CE = C * E
NCHUNK = TW // C
G = C // L

_mesh = plsc.VectorSubcoreMesh(
    core_axis_name="c", subcore_axis_name="s", num_cores=NC, num_subcores=NS
)


@functools.partial(
    pl.kernel,
    out_type=(
        jax.ShapeDtypeStruct((W, E, L), jnp.float32),
        jax.ShapeDtypeStruct((W, E, L), jnp.float32),
    ),
    mesh=_mesh,
    compiler_params=pltpu.CompilerParams(
        needs_layout_passes=False, use_tc_tiling_on_sc=False
    ),
    scratch_types=[
        pltpu.VMEM((CE,), jnp.float32),
        pltpu.VMEM((CE,), jnp.float32),
        pltpu.VMEM((E, L), jnp.float32),
        pltpu.VMEM((E, L), jnp.float32),
        pltpu.VMEM((E, L), jnp.float32),
        pltpu.SemaphoreType.DMA,
        pltpu.SemaphoreType.DMA,
    ],
)
def _sc_balance(gate_hbm, w_out, hit_out, buf0, buf1, evbuf, accw, acchit,
                sem0, sem1):
    wid = lax.axis_index("s") * NC + lax.axis_index("c")
    base = wid * (TW * E)
    iota = lax.iota(jnp.int32, L)
    zero_v = jnp.zeros((L,), jnp.float32)
    neg_inf = jnp.full((L,), -jnp.inf, dtype=jnp.float32)
    bufs = (buf0, buf1)
    sems = (sem0, sem1)

    def chunk_copy(c_idx, b):
        off = base + jnp.minimum(c_idx, NCHUNK - 1) * CE
        return pltpu.make_async_copy(
            gate_hbm.at[pl.ds(off, CE)], bufs[b], sems[b]
        )

    @plsc.parallel_loop(0, E, 1, unroll=8)
    def _init(e):
        accw[e, :] = zero_v
        acchit[e, :] = zero_v

    chunk_copy(0, 0).start()
    chunk_copy(1, 1).start()

    def process(buf):
        def group_body(g, _):
            idx0 = g * (L * E) + iota * E

            @plsc.parallel_loop(0, E, 1, unroll=8, carry=(zero_v, neg_inf, neg_inf))
            def p1(e, carry):
                s, m1, m2 = carry
                v = plsc.load_gather(buf, [idx0 + ((iota + e) & (E - 1))])
                ev = jnp.exp(v)
                evbuf[e, :] = ev
                m2 = jnp.maximum(m2, jnp.minimum(m1, ev))
                m1 = jnp.maximum(m1, ev)
                return s + ev, m1, m2

            s, _, m2 = p1
            r = 1.0 / s

            @plsc.parallel_loop(0, E, 1, unroll=8)
            def _p2(e):
                ev = evbuf[e, :]
                accw[e, :] = accw[e, :] + ev * r
                h = jnp.where(ev >= m2, 1.0, 0.0).astype(jnp.float32)
                acchit[e, :] = jnp.maximum(acchit[e, :], h)

            return 0

        lax.fori_loop(0, G, group_body, 0)

    def chunk_body(cc, _):
        for b in range(2):
            c = cc * 2 + b
            chunk_copy(c, b).wait()
            process(bufs[b])
            chunk_copy(c + 2, b).start()
        return 0

    lax.fori_loop(0, NCHUNK // 2, chunk_body, 0)
    chunk_copy(NCHUNK, 0).wait()
    chunk_copy(NCHUNK + 1, 1).wait()

    pltpu.sync_copy(accw, w_out.at[wid])
    pltpu.sync_copy(acchit, hit_out.at[wid])


def kernel(gate_logits):
    w_parts, hit_parts = _sc_balance(gate_logits.reshape(-1))
    total_w = jnp.sum(w_parts)
    true_e = (jnp.arange(E)[:, None] + jnp.arange(L)[None, :]) % E
    hit_el = jnp.max(hit_parts, axis=0)
    hit_any = jnp.zeros((E,), jnp.float32).at[true_e].max(hit_el) > 0.5
    tpe = jnp.mean(hit_any.astype(jnp.float32))
    return tpe * (total_w / np.float32(N_TOK)) * np.float32(E)


# speedup vs baseline: 3.0420x; 1.0006x over previous
import functools

import numpy as np
import jax
import jax.numpy as jnp
from jax import lax
from jax.experimental import pallas as pl
from jax.experimental.pallas import tpu as pltpu
from jax.experimental.pallas import tpu_sc as plsc

E = 64
L = 16
N_TOK = 131072
NC, NS = 2, 16
W = NC * NS
TW = N_TOK // W
C = 512
CE = C * E
NCHUNK = TW // C
G = C // L

_mesh = plsc.VectorSubcoreMesh(
    core_axis_name="c", subcore_axis_name="s", num_cores=NC, num_subcores=NS
)


@functools.partial(
    pl.kernel,
    out_type=(
        jax.ShapeDtypeStruct((W, E, L), jnp.float32),
        jax.ShapeDtypeStruct((W, E, L), jnp.float32),
    ),
    mesh=_mesh,
    compiler_params=pltpu.CompilerParams(
        needs_layout_passes=False, use_tc_tiling_on_sc=False
    ),
    scratch_types=[
        pltpu.VMEM((CE,), jnp.float32),
        pltpu.VMEM((CE,), jnp.float32),
        pltpu.VMEM((E, L), jnp.float32),
        pltpu.VMEM((E, L), jnp.float32),
        pltpu.VMEM((E, L), jnp.float32),
        pltpu.SemaphoreType.DMA,
        pltpu.SemaphoreType.DMA,
    ],
)
def _sc_balance(gate_hbm, w_out, hit_out, buf0, buf1, evbuf, accw, acchit,
                sem0, sem1):
    wid = lax.axis_index("s") * NC + lax.axis_index("c")
    base = wid * (TW * E)
    iota = lax.iota(jnp.int32, L)
    zero_v = jnp.zeros((L,), jnp.float32)
    neg_inf = jnp.full((L,), -jnp.inf, dtype=jnp.float32)
    bufs = (buf0, buf1)
    sems = (sem0, sem1)

    def chunk_copy(c_idx, b):
        off = base + jnp.minimum(c_idx, NCHUNK - 1) * CE
        return pltpu.make_async_copy(
            gate_hbm.at[pl.ds(off, CE)], bufs[b], sems[b]
        )

    @plsc.parallel_loop(0, E, 1, unroll=8)
    def _init(e):
        accw[e, :] = zero_v
        acchit[e, :] = zero_v

    chunk_copy(0, 0).start()
    chunk_copy(1, 1).start()

    def process(buf):
        def group_body(g, _):
            idx0 = g * (L * E) + iota * E

            @plsc.parallel_loop(0, E, 1, unroll=8, carry=(zero_v, neg_inf, neg_inf))
            def p1(e, carry):
                s, m1, m2 = carry
                v = plsc.load_gather(buf, [idx0 + ((iota + e) & (E - 1))])
                ev = jnp.exp(v)
                evbuf[e, :] = ev
                m2 = jnp.maximum(m2, jnp.minimum(m1, ev))
                m1 = jnp.maximum(m1, ev)
                return s + ev, m1, m2

            s, _, m2 = p1
            r = 1.0 / s

            @plsc.parallel_loop(0, E, 1, unroll=8)
            def _p2(e):
                ev = evbuf[e, :]
                plsc.addupdate(accw.at[e], ev * r)
                plsc.addupdate(acchit.at[e], jnp.where(ev >= m2, 1.0, 0.0))

            return 0

        lax.fori_loop(0, G, group_body, 0)

    def chunk_body(cc, _):
        for b in range(2):
            c = cc * 2 + b
            chunk_copy(c, b).wait()
            process(bufs[b])
            chunk_copy(c + 2, b).start()
        return 0

    lax.fori_loop(0, NCHUNK // 2, chunk_body, 0)
    chunk_copy(NCHUNK, 0).wait()
    chunk_copy(NCHUNK + 1, 1).wait()

    pltpu.sync_copy(accw, w_out.at[wid])
    pltpu.sync_copy(acchit, hit_out.at[wid])


def kernel(gate_logits):
    w_parts, hit_parts = _sc_balance(gate_logits.reshape(-1))
    total_w = jnp.sum(w_parts)
    true_e = (jnp.arange(E)[:, None] + jnp.arange(L)[None, :]) % E
    hit_el = jnp.max(hit_parts, axis=0)
    hit_any = jnp.zeros((E,), jnp.float32).at[true_e].max(hit_el) > 0.5
    tpe = jnp.mean(hit_any.astype(jnp.float32))
    return tpe * (total_w / np.float32(N_TOK)) * np.float32(E)


# speedup vs baseline: 3.7686x; 1.2388x over previous
import functools

import numpy as np
import jax
import jax.numpy as jnp
from jax import lax
from jax.experimental import pallas as pl
from jax.experimental.pallas import tpu as pltpu
from jax.experimental.pallas import tpu_sc as plsc

E = 64
L = 16
N_TOK = 131072
NC, NS = 2, 16
W = NC * NS
TW = N_TOK // W
C = 512
NCHUNK = TW // C
G = C // L

_mesh = plsc.VectorSubcoreMesh(
    core_axis_name="c", subcore_axis_name="s", num_cores=NC, num_subcores=NS
)


@functools.partial(
    pl.kernel,
    out_type=(
        jax.ShapeDtypeStruct((W, E), jnp.float32),
        jax.ShapeDtypeStruct((W, E), jnp.float32),
    ),
    mesh=_mesh,
    compiler_params=pltpu.CompilerParams(
        needs_layout_passes=False, use_tc_tiling_on_sc=False
    ),
    scratch_types=[
        pltpu.VMEM((C, E), jnp.float32),
        pltpu.VMEM((C, E), jnp.float32),
        pltpu.VMEM((E, L), jnp.float32),
        pltpu.VMEM((E,), jnp.float32),
        pltpu.VMEM((E,), jnp.float32),
        pltpu.SemaphoreType.DMA,
        pltpu.SemaphoreType.DMA,
    ],
)
def _sc_balance(gate_hbm, w_out, hit_out, buf0, buf1, evbuf, accw, acchit,
                sem0, sem1):
    wid = lax.axis_index("s") * NC + lax.axis_index("c")
    tok0 = wid * TW
    iota = lax.iota(jnp.int32, L)
    zero_v = jnp.zeros((L,), jnp.float32)
    neg_inf = jnp.full((L,), -jnp.inf, dtype=jnp.float32)
    bufs = (buf0, buf1)
    sems = (sem0, sem1)

    def chunk_copy(c_idx, b):
        row = tok0 + jnp.minimum(c_idx, NCHUNK - 1) * C
        return pltpu.make_async_copy(
            gate_hbm.at[pl.ds(row, C)], bufs[b], sems[b]
        )

    for e in range(0, E, L):
        accw[pl.ds(e, L)] = zero_v
        acchit[pl.ds(e, L)] = zero_v

    chunk_copy(0, 0).start()
    chunk_copy(1, 1).start()

    def process(buf):
        def group_body(g, _):
            rows = g * L + iota

            @plsc.parallel_loop(0, E, 1, unroll=8, carry=(zero_v, neg_inf, neg_inf))
            def p1(e, carry):
                s, m1, m2 = carry
                col = (iota + e) & (E - 1)
                v = plsc.load_gather(buf, [rows, col])
                ev = jnp.exp(v)
                evbuf[e, :] = ev
                m2 = jnp.maximum(m2, jnp.minimum(m1, ev))
                m1 = jnp.maximum(m1, ev)
                return s + ev, m1, m2

            s, _, m2 = p1
            r = 1.0 / s

            @plsc.parallel_loop(0, E, 1, unroll=8)
            def _p2(e):
                ev = evbuf[e, :]
                col = (iota + e) & (E - 1)
                plsc.addupdate_scatter(accw, [col], ev * r)
                plsc.addupdate_scatter(acchit, [col], jnp.where(ev >= m2, 1.0, 0.0))

            return 0

        lax.fori_loop(0, G, group_body, 0)

    def chunk_body(cc, _):
        for b in range(2):
            c = cc * 2 + b
            chunk_copy(c, b).wait()
            process(bufs[b])
            chunk_copy(c + 2, b).start()
        return 0

    lax.fori_loop(0, NCHUNK // 2, chunk_body, 0)
    chunk_copy(NCHUNK, 0).wait()
    chunk_copy(NCHUNK + 1, 1).wait()

    pltpu.sync_copy(accw, w_out.at[wid])
    pltpu.sync_copy(acchit, hit_out.at[wid])


def kernel(gate_logits):
    w_parts, hit_parts = _sc_balance(gate_logits)
    total_w = jnp.sum(w_parts)
    hit_any = jnp.max(hit_parts, axis=0) > 0.5
    tpe = jnp.mean(hit_any.astype(jnp.float32))
    return tpe * (total_w / np.float32(N_TOK)) * np.float32(E)


# speedup vs baseline: 3.7714x; 1.0007x over previous
import functools

import numpy as np
import jax
import jax.numpy as jnp
from jax import lax
from jax.experimental import pallas as pl
from jax.experimental.pallas import tpu as pltpu
from jax.experimental.pallas import tpu_sc as plsc

E = 64
L = 16
N_TOK = 131072
NC, NS = 2, 16
W = NC * NS
TW = N_TOK // W
C = 512
NCHUNK = TW // C
G = C // L

_mesh = plsc.VectorSubcoreMesh(
    core_axis_name="c", subcore_axis_name="s", num_cores=NC, num_subcores=NS
)


@functools.partial(
    pl.kernel,
    out_type=(
        jax.ShapeDtypeStruct((W, E), jnp.float32),
        jax.ShapeDtypeStruct((W, E), jnp.float32),
    ),
    mesh=_mesh,
    compiler_params=pltpu.CompilerParams(
        needs_layout_passes=False, use_tc_tiling_on_sc=False
    ),
    scratch_types=[
        pltpu.VMEM((C // 2, 2 * E), jnp.float32),
        pltpu.VMEM((C // 2, 2 * E), jnp.float32),
        pltpu.VMEM((E, L), jnp.float32),
        pltpu.VMEM((E,), jnp.float32),
        pltpu.VMEM((E,), jnp.float32),
        pltpu.SemaphoreType.DMA,
        pltpu.SemaphoreType.DMA,
    ],
)
def _sc_balance(gate_hbm, w_out, hit_out, buf0, buf1, evbuf, accw, acchit,
                sem0, sem1):
    wid = lax.axis_index("s") * NC + lax.axis_index("c")
    tok0 = wid * TW
    iota = lax.iota(jnp.int32, L)
    zero_v = jnp.zeros((L,), jnp.float32)
    neg_inf = jnp.full((L,), -jnp.inf, dtype=jnp.float32)
    bufs = (buf0, buf1)
    sems = (sem0, sem1)

    def chunk_copy(c_idx, b):
        row = (tok0 + jnp.minimum(c_idx, NCHUNK - 1) * C) // 2
        return pltpu.make_async_copy(
            gate_hbm.at[pl.ds(row, C // 2)], bufs[b], sems[b]
        )

    for e in range(0, E, L):
        accw[pl.ds(e, L)] = zero_v
        acchit[pl.ds(e, L)] = zero_v

    chunk_copy(0, 0).start()
    chunk_copy(1, 1).start()

    iota_half = iota >> 1
    colpar = (iota & 1) << 6

    def process(buf):
        def group_body(g, _):
            rows = g * (L // 2) + iota_half

            @plsc.parallel_loop(0, E, 1, unroll=8, carry=(zero_v, neg_inf, neg_inf))
            def p1(e, carry):
                s, m1, m2 = carry
                col = colpar + ((iota + e) & (E - 1))
                v = plsc.load_gather(buf, [rows, col])
                ev = jnp.exp(v)
                evbuf[e, :] = ev
                m2 = jnp.maximum(m2, jnp.minimum(m1, ev))
                m1 = jnp.maximum(m1, ev)
                return s + ev, m1, m2

            s, _, m2 = p1
            r = 1.0 / s

            @plsc.parallel_loop(0, E, 1, unroll=8)
            def _p2(e):
                ev = evbuf[e, :]
                col = (iota + e) & (E - 1)
                plsc.addupdate_scatter(accw, [col], ev * r)
                plsc.addupdate_scatter(acchit, [col], jnp.where(ev >= m2, 1.0, 0.0))

            return 0

        lax.fori_loop(0, G, group_body, 0)

    def chunk_body(cc, _):
        for b in range(2):
            c = cc * 2 + b
            chunk_copy(c, b).wait()
            process(bufs[b])
            chunk_copy(c + 2, b).start()
        return 0

    lax.fori_loop(0, NCHUNK // 2, chunk_body, 0)
    chunk_copy(NCHUNK, 0).wait()
    chunk_copy(NCHUNK + 1, 1).wait()

    pltpu.sync_copy(accw, w_out.at[wid])
    pltpu.sync_copy(acchit, hit_out.at[wid])


def kernel(gate_logits):
    w_parts, hit_parts = _sc_balance(gate_logits.reshape(N_TOK // 2, 2 * E))
    total_w = jnp.sum(w_parts)
    hit_any = jnp.max(hit_parts, axis=0) > 0.5
    tpe = jnp.mean(hit_any.astype(jnp.float32))
    return tpe * (total_w / np.float32(N_TOK)) * np.float32(E)
